# baseline (speedup 1.0000x reference)
import functools

import jax
import jax.numpy as jnp
from jax import lax
from jax.experimental import pallas as pl
from jax.experimental.pallas import tpu as pltpu
from jax.experimental.pallas import tpu_sc as plsc

NC = 2
NS = 16
NW = NC * NS
L = 16

B = 16384
D = 32
BPW = B // NW
CHUNK = 128
NCHUNK = BPW // CHUNK
GROUPS = BPW // L


def _body(uid_hbm, iid_hbm, ut_hbm, it_hbm, out_hbm,
          idx_u, idx_i, u_rows, i_rows, out_v, sem_u, sem_i):
    wid = lax.axis_index("s") * NC + lax.axis_index("c")
    base = wid * BPW

    pltpu.sync_copy(uid_hbm.at[wid], idx_u)
    pltpu.sync_copy(iid_hbm.at[wid], idx_i)

    copies = []
    for j in range(NCHUNK):
        copies.append(pltpu.async_copy(
            ut_hbm.at[idx_u.at[j]], u_rows.at[pl.ds(j * CHUNK, CHUNK)], sem_u))
        copies.append(pltpu.async_copy(
            it_hbm.at[idx_i.at[j]], i_rows.at[pl.ds(j * CHUNK, CHUNK)], sem_i))
    for cp in copies:
        cp.wait()

    iota = lax.iota(jnp.int32, L)

    def group(g, carry):
        base_row = g * L
        vals = jnp.zeros((L,), jnp.float32)
        for r in range(L):
            row = base_row + r
            prod = (u_rows[row, pl.ds(0, L)] * i_rows[row, pl.ds(0, L)]
                    + u_rows[row, pl.ds(L, L)] * i_rows[row, pl.ds(L, L)])
            vals = jnp.where(iota == r, jnp.sum(prod), vals)
        out_v[pl.ds(base_row, L)] = vals
        return carry

    lax.fori_loop(0, GROUPS, group, None)

    pltpu.sync_copy(out_v, out_hbm.at[pl.ds(base, BPW)])


@functools.cache
def _build():
    return pl.kernel(
        _body,
        out_type=jax.ShapeDtypeStruct((B,), jnp.float32),
        mesh=plsc.VectorSubcoreMesh(core_axis_name="c", subcore_axis_name="s",
                                    num_cores=NC, num_subcores=NS),
        compiler_params=pltpu.CompilerParams(needs_layout_passes=False,
                                             use_tc_tiling_on_sc=False),
        scratch_types=[
            pltpu.VMEM((NCHUNK, CHUNK), jnp.int32),
            pltpu.VMEM((NCHUNK, CHUNK), jnp.int32),
            pltpu.VMEM((BPW, D), jnp.float32),
            pltpu.VMEM((BPW, D), jnp.float32),
            pltpu.VMEM((BPW,), jnp.float32),
            pltpu.SemaphoreType.DMA,
            pltpu.SemaphoreType.DMA,
        ],
    )


@jax.jit
def kernel(user_id, item_id, user_table, item_table):
    uid = user_id.astype(jnp.int32).reshape(NW, NCHUNK, CHUNK)
    iid = item_id.astype(jnp.int32).reshape(NW, NCHUNK, CHUNK)
    return _build()(uid, iid, user_table, item_table)
